# baseline (speedup 1.0000x reference)
import jax
import jax.numpy as jnp
from jax.experimental import pallas as pl
from jax.experimental.pallas import tpu as pltpu


def _gcn_body(p_ref, w1_ref, b1_ref, w2_ref, b2_ref, w3_ref, b3_ref,
              ones_ref, out_ref):
    f32 = jnp.float32
    bf16 = jnp.bfloat16
    bs = p_ref.shape[0]

    def sample(i):
        p = p_ref[i]
        pt = p.T
        px_c = p[:, 0:1]
        py_c = p[:, 1:2]
        px_r = pt[0:1, :]
        py_r = pt[1:2, :]

        dx = px_c - px_r
        dy = py_c - py_r
        a = (dx * dx + dy * dy < 1.0).astype(bf16)

        deg_c = jnp.dot(a, ones_ref[...], preferred_element_type=f32)
        r_c = jax.lax.rsqrt(deg_c)

        def agg(x):
            y = jnp.dot(a, (x * r_c).astype(bf16), preferred_element_type=f32)
            return y * r_c

        xw1 = px_c * w1_ref[0:1, :] + py_c * w1_ref[1:2, :]
        h1 = jax.nn.relu(agg(xw1) + b1_ref[0:1, :])
        xw2 = jnp.dot(h1.astype(bf16), w2_ref[...], preferred_element_type=f32)
        h2 = jax.nn.relu(agg(xw2) + b2_ref[0:1, :])
        xw3 = jnp.dot(h2.astype(bf16), w3_ref[...], preferred_element_type=f32)
        out_ref[i] = agg(xw3) + b3_ref[0:1, :]

    for i in range(bs):
        sample(i)


def kernel(points, W1, b1, W2, b2, W3, b3):
    bs, n, _ = points.shape
    d3 = W3.shape[1]
    ones = jnp.ones((n, 1), jnp.bfloat16)
    return pl.pallas_call(
        _gcn_body,
        out_shape=jax.ShapeDtypeStruct((bs, n, d3), jnp.float32),
    )(points, W1, b1.reshape(1, -1), W2, b2.reshape(1, -1),
      W3, b3.reshape(1, -1), ones)


kernel = jax.jit(kernel)


# speedup vs baseline: 1.0317x; 1.0317x over previous
import jax
import jax.numpy as jnp
from jax.experimental import pallas as pl
from jax.experimental.pallas import tpu as pltpu


def _gcn_body(p_ref, w1_ref, b1_ref, w2_ref, b2_ref, w3_ref, b3_ref,
              ones_ref, out_ref):
    f32 = jnp.float32
    bf16 = jnp.bfloat16
    p = p_ref[0]
    pt = p.T
    px_c = p[:, 0:1]
    py_c = p[:, 1:2]
    px_r = pt[0:1, :]
    py_r = pt[1:2, :]

    dx = px_c - px_r
    dy = py_c - py_r
    a = (dx * dx + dy * dy < 1.0).astype(bf16)

    deg_c = jnp.dot(a, ones_ref[...], preferred_element_type=f32)
    r_c = jax.lax.rsqrt(deg_c)

    def agg(x):
        y = jnp.dot(a, (x * r_c).astype(bf16), preferred_element_type=f32)
        return y * r_c

    xw1 = px_c * w1_ref[0:1, :] + py_c * w1_ref[1:2, :]
    h1 = jax.nn.relu(agg(xw1) + b1_ref[0:1, :])
    xw2 = jnp.dot(h1.astype(bf16), w2_ref[...], preferred_element_type=f32)
    h2 = jax.nn.relu(agg(xw2) + b2_ref[0:1, :])
    xw3 = jnp.dot(h2.astype(bf16), w3_ref[...], preferred_element_type=f32)
    out_ref[0] = agg(xw3) + b3_ref[0:1, :]


def kernel(points, W1, b1, W2, b2, W3, b3):
    bs, n, _ = points.shape
    d3 = W3.shape[1]
    ones = jnp.ones((n, 1), jnp.bfloat16)
    full = lambda shape: pl.BlockSpec(shape, lambda i: (0,) * len(shape))
    return pl.pallas_call(
        _gcn_body,
        grid=(bs,),
        in_specs=[
            pl.BlockSpec((1, n, 2), lambda i: (i, 0, 0)),
            full(W1.shape),
            full((1, b1.shape[0])),
            full(W2.shape),
            full((1, b2.shape[0])),
            full(W3.shape),
            full((1, b3.shape[0])),
            full((n, 1)),
        ],
        out_specs=pl.BlockSpec((1, n, d3), lambda i: (i, 0, 0)),
        out_shape=jax.ShapeDtypeStruct((bs, n, d3), jnp.float32),
        compiler_params=pltpu.CompilerParams(
            dimension_semantics=("arbitrary",)),
    )(points, W1, b1.reshape(1, -1), W2, b2.reshape(1, -1),
      W3, b3.reshape(1, -1), ones)


kernel = jax.jit(kernel)


# speedup vs baseline: 1.0822x; 1.0490x over previous
import functools

import jax
import jax.numpy as jnp
import numpy as np
from jax.experimental import pallas as pl
from jax.experimental.pallas import tpu as pltpu
from jax.sharding import Mesh, PartitionSpec as P


def _gcn_body(p_ref, w1_ref, b1_ref, w2_ref, b2_ref, w3_ref, b3_ref,
              out_ref):
    f32 = jnp.float32
    bf16 = jnp.bfloat16
    p = p_ref[0]
    pt = p.T
    px_c = p[:, 0:1]
    py_c = p[:, 1:2]
    px_r = pt[0:1, :]
    py_r = pt[1:2, :]

    dx = px_c - px_r
    dy = py_c - py_r
    af = (dx * dx + dy * dy < 1.0).astype(f32)
    a = af.astype(bf16)

    deg_c = jnp.sum(af, axis=1, keepdims=True)
    r_c = jax.lax.rsqrt(deg_c)

    def agg(xb, b):
        y = jnp.dot(a, xb, preferred_element_type=f32)
        return y * r_c + b[0:1, :]

    pxs = px_c * r_c
    pys = py_c * r_c
    xw1 = (pxs * w1_ref[0:1, :] + pys * w1_ref[1:2, :]).astype(bf16)
    h1 = jax.nn.relu(agg(xw1, b1_ref)).astype(bf16)
    xw2 = (jnp.dot(h1, w2_ref[...], preferred_element_type=f32)
           * r_c).astype(bf16)
    h2 = jax.nn.relu(agg(xw2, b2_ref)).astype(bf16)
    xw3 = (jnp.dot(h2, w3_ref[...], preferred_element_type=f32)
           * r_c).astype(bf16)
    out_ref[0] = agg(xw3, b3_ref)


def _gcn_shard(points, W1, b1, W2, b2, W3, b3):
    bs, n, _ = points.shape
    d3 = W3.shape[1]
    full = lambda shape: pl.BlockSpec(shape, lambda i: (0,) * len(shape))
    return pl.pallas_call(
        _gcn_body,
        grid=(bs,),
        in_specs=[
            pl.BlockSpec((1, n, 2), lambda i: (i, 0, 0)),
            full(W1.shape),
            full((1, b1.shape[0])),
            full(W2.shape),
            full((1, b2.shape[0])),
            full(W3.shape),
            full((1, b3.shape[0])),
        ],
        out_specs=pl.BlockSpec((1, n, d3), lambda i: (i, 0, 0)),
        out_shape=jax.ShapeDtypeStruct((bs, n, d3), jnp.float32),
        compiler_params=pltpu.CompilerParams(
            dimension_semantics=("parallel",)),
    )(points, W1, b1.reshape(1, -1), W2, b2.reshape(1, -1),
      W3, b3.reshape(1, -1))


def kernel(points, W1, b1, W2, b2, W3, b3):
    return _gcn_shard(points, W1, b1, W2, b2, W3, b3)


kernel = jax.jit(kernel)


# speedup vs baseline: 1.0939x; 1.0108x over previous
import functools

import jax
import jax.numpy as jnp
import numpy as np
from jax.experimental import pallas as pl
from jax.experimental.pallas import tpu as pltpu
from jax.sharding import Mesh, PartitionSpec as P


def _gcn_body(p_ref, w1_ref, b1_ref, w2_ref, b2_ref, w3_ref, b3_ref,
              out_ref):
    f32 = jnp.float32
    bf16 = jnp.bfloat16
    p = p_ref[0]
    pt = p.T
    px_c = p[:, 0:1]
    py_c = p[:, 1:2]
    px_r = pt[0:1, :]
    py_r = pt[1:2, :]

    dx = px_c - px_r
    dy = py_c - py_r
    af = (dx * dx + dy * dy < 1.0).astype(f32)
    a = af.astype(bf16)

    deg_c = jnp.sum(af, axis=1, keepdims=True)
    r_c = jax.lax.rsqrt(deg_c)

    def agg(x):
        y = jnp.dot(a, (x * r_c).astype(bf16), preferred_element_type=f32)
        return y * r_c

    xw1 = px_c * w1_ref[0:1, :] + py_c * w1_ref[1:2, :]
    h1 = jax.nn.relu(agg(xw1) + b1_ref[0:1, :])
    xw2 = jnp.dot(h1.astype(bf16), w2_ref[...], preferred_element_type=f32)
    h2 = jax.nn.relu(agg(xw2) + b2_ref[0:1, :])
    xw3 = jnp.dot(h2.astype(bf16), w3_ref[...], preferred_element_type=f32)
    out_ref[0] = agg(xw3) + b3_ref[0:1, :]


def _gcn_shard(points, W1, b1, W2, b2, W3, b3):
    bs, n, _ = points.shape
    d3 = W3.shape[1]
    full = lambda shape: pl.BlockSpec(shape, lambda i: (0,) * len(shape))
    return pl.pallas_call(
        _gcn_body,
        grid=(bs,),
        in_specs=[
            pl.BlockSpec((1, n, 2), lambda i: (i, 0, 0)),
            full(W1.shape),
            full((1, b1.shape[0])),
            full(W2.shape),
            full((1, b2.shape[0])),
            full(W3.shape),
            full((1, b3.shape[0])),
        ],
        out_specs=pl.BlockSpec((1, n, d3), lambda i: (i, 0, 0)),
        out_shape=jax.ShapeDtypeStruct((bs, n, d3), jnp.float32),
        compiler_params=pltpu.CompilerParams(
            dimension_semantics=("parallel",)),
    )(points, W1, b1.reshape(1, -1), W2, b2.reshape(1, -1),
      W3, b3.reshape(1, -1))


def kernel(points, W1, b1, W2, b2, W3, b3):
    return _gcn_shard(points, W1, b1, W2, b2, W3, b3)


kernel = jax.jit(kernel)


# speedup vs baseline: 1.0950x; 1.0010x over previous
import jax
import jax.numpy as jnp
from jax.experimental import pallas as pl
from jax.experimental.pallas import tpu as pltpu


def _gcn_body(p_ref, w1_ref, b1_ref, w2_ref, b2_ref, w3_ref, b3_ref,
              out_ref):
    f32 = jnp.float32
    bf16 = jnp.bfloat16
    p = p_ref[0]
    pt = p.T
    px_c = p[:, 0:1]
    py_c = p[:, 1:2]
    px_r = pt[0:1, :]
    py_r = pt[1:2, :]

    dx = px_c - px_r
    dy = py_c - py_r
    af = (dx * dx + dy * dy < 1.0).astype(f32)
    a = af.astype(bf16)

    deg_c = jnp.sum(af, axis=1, keepdims=True)
    r_c = jax.lax.rsqrt(deg_c)

    def agg(x):
        y = jnp.dot(a, (x * r_c).astype(bf16), preferred_element_type=f32)
        return y * r_c

    xw1 = px_c * w1_ref[0:1, :] + py_c * w1_ref[1:2, :]
    h1 = jax.nn.relu(agg(xw1) + b1_ref[0:1, :])
    xw2 = jnp.dot(h1.astype(bf16), w2_ref[...], preferred_element_type=f32)
    h2 = jax.nn.relu(agg(xw2) + b2_ref[0:1, :])
    xw3 = jnp.dot(h2.astype(bf16), w3_ref[...], preferred_element_type=f32)
    out_ref[0] = agg(xw3) + b3_ref[0:1, :]


def _gcn_call(points, W1, b1, W2, b2, W3, b3):
    bs, n, _ = points.shape
    d3 = W3.shape[1]
    full = lambda shape: pl.BlockSpec(shape, lambda i: (0,) * len(shape))
    return pl.pallas_call(
        _gcn_body,
        grid=(bs,),
        in_specs=[
            pl.BlockSpec((1, n, 2), lambda i: (i, 0, 0)),
            full(W1.shape),
            full((1, b1.shape[0])),
            full(W2.shape),
            full((1, b2.shape[0])),
            full(W3.shape),
            full((1, b3.shape[0])),
        ],
        out_specs=pl.BlockSpec((1, n, d3), lambda i: (i, 0, 0)),
        out_shape=jax.ShapeDtypeStruct((bs, n, d3), jnp.float32),
        compiler_params=pltpu.CompilerParams(
            dimension_semantics=("parallel",)),
    )(points, W1, b1.reshape(1, -1), W2, b2.reshape(1, -1),
      W3, b3.reshape(1, -1))


def kernel(points, W1, b1, W2, b2, W3, b3):
    return _gcn_call(points, W1, b1, W2, b2, W3, b3)


kernel = jax.jit(kernel)


# speedup vs baseline: 1.1023x; 1.0067x over previous
import jax
import jax.numpy as jnp
from jax.experimental import pallas as pl
from jax.experimental.pallas import tpu as pltpu


def _gcn_body(p_ref, w1_ref, b1_ref, w2_ref, b2_ref, w3_ref, b3_ref,
              out_ref):
    f32 = jnp.float32
    bf16 = jnp.bfloat16
    p = p_ref[0]
    pt = p.T
    px_c = p[:, 0:1]
    py_c = p[:, 1:2]
    px_r = pt[0:1, :]
    py_r = pt[1:2, :]

    dx = px_c - px_r
    dy = py_c - py_r
    af = (dx * dx + dy * dy < 1.0).astype(f32)
    a = af

    deg_c = jnp.sum(af, axis=1, keepdims=True)
    r_c = jax.lax.rsqrt(deg_c)

    def agg(x):
        y = jnp.dot(a, (x * r_c).astype(bf16), preferred_element_type=f32)
        return y * r_c

    xw1 = px_c * w1_ref[0:1, :] + py_c * w1_ref[1:2, :]
    h1 = jax.nn.relu(agg(xw1) + b1_ref[0:1, :])
    xw2 = jnp.dot(h1.astype(bf16), w2_ref[...], preferred_element_type=f32)
    h2 = jax.nn.relu(agg(xw2) + b2_ref[0:1, :])
    xw3 = jnp.dot(h2.astype(bf16), w3_ref[...], preferred_element_type=f32)
    out_ref[0] = agg(xw3) + b3_ref[0:1, :]


def _gcn_call(points, W1, b1, W2, b2, W3, b3):
    bs, n, _ = points.shape
    d3 = W3.shape[1]
    full = lambda shape: pl.BlockSpec(shape, lambda i: (0,) * len(shape))
    return pl.pallas_call(
        _gcn_body,
        grid=(bs,),
        in_specs=[
            pl.BlockSpec((1, n, 2), lambda i: (i, 0, 0)),
            full(W1.shape),
            full((1, b1.shape[0])),
            full(W2.shape),
            full((1, b2.shape[0])),
            full(W3.shape),
            full((1, b3.shape[0])),
        ],
        out_specs=pl.BlockSpec((1, n, d3), lambda i: (i, 0, 0)),
        out_shape=jax.ShapeDtypeStruct((bs, n, d3), jnp.float32),
        compiler_params=pltpu.CompilerParams(
            dimension_semantics=("parallel",)),
    )(points, W1, b1.reshape(1, -1), W2, b2.reshape(1, -1),
      W3, b3.reshape(1, -1))


def kernel(points, W1, b1, W2, b2, W3, b3):
    return _gcn_call(points, W1, b1, W2, b2, W3, b3)


kernel = jax.jit(kernel)
